# baseline (speedup 1.0000x reference)
import numpy as np
import jax
import jax.numpy as jnp
from jax.experimental import pallas as pl
from jax.experimental.pallas import tpu as pltpu

P = 8
N = 10000
EMBED = 128
H = 2
C = H * EMBED
GOUT = H * C
HID = 4 * EMBED
OUTD = 128

BLOCK = 2000
W = 32
NEG = -1e30

_S_np = np.zeros((W, 8), np.float32)
_S_np[0:9, 0] = 1.0
_S_np[16:25, 1] = 1.0
_ST_np = np.zeros((8, W), np.float32)
_ST_np[0, 0:9] = 1.0
_ST_np[1, 16:25] = 1.0
_E_np = np.zeros((W, P), np.float32)
_E_np[0:8, :] = np.eye(P, dtype=np.float32)
_E_np[16:24, :] = np.eye(P, dtype=np.float32)
_HPmask_np = np.zeros((W, GOUT), np.float32)
_HPmask_np[0:8, :C] = 1.0
_HPmask_np[16:24, C:] = 1.0
_PAD_np = np.full((1, W), NEG, np.float32)
_PAD_np[0, 0:9] = 0.0
_PAD_np[0, 16:25] = 0.0
_G0_np = np.zeros((P, W), np.float32)
_G0_np[:, 0:8] = np.eye(P, dtype=np.float32)
_G1_np = np.zeros((P, W), np.float32)
_G1_np[:, 16:24] = np.eye(P, dtype=np.float32)
_M0_np = np.zeros((1, GOUT), np.float32)
_M0_np[0, :C] = 1.0
_R4_np = np.zeros((4, W), np.float32)
_R4_np[0, 0:8] = 1.0
_R4_np[1, 8] = 1.0
_R4_np[2, 16:24] = 1.0
_R4_np[3, 24] = 1.0


def _fused_kernel(x_ref, p_ref, wg_ref, asrc_ref, adst_ref, m0_ref, r4_ref,
                  g0_ref, g1_ref, pad_ref, s_ref, st_ref, e_ref,
                  hpm_ref, bg_ref,
                  w1_ref, b1_ref, w2_ref, b2_ref, wf_ref, bf_ref,
                  preds_ref, f_ref,
                  wgm2_scr, hpw2_scr, apvec_scr):
    f32 = jnp.float32
    dn = (((1,), (1,)), ((), ()))

    @pl.when(pl.program_id(0) == 0)
    def _prep():
        wg = wg_ref[...]
        asrc = asrc_ref[...]
        adst = adst_ref[...]
        m0 = m0_ref[...]
        m1 = 1.0 - m0
        cols = jnp.concatenate([adst * m0, (asrc + adst) * m0,
                                adst * m1, (asrc + adst) * m1], axis=0)
        w4 = jax.lax.dot_general(wg, cols, dn, preferred_element_type=f32)
        wgm2_scr[...] = jnp.dot(w4, r4_ref[...], preferred_element_type=f32)

        hp = jnp.dot(p_ref[...], wg, preferred_element_type=f32)
        ap0 = jax.lax.dot_general(asrc * m0, hp, dn, preferred_element_type=f32)
        ap1 = jax.lax.dot_general(asrc * m1, hp, dn, preferred_element_type=f32)
        apvec_scr[...] = (jnp.dot(ap0, g0_ref[...], preferred_element_type=f32)
                          + jnp.dot(ap1, g1_ref[...], preferred_element_type=f32)
                          + pad_ref[...])
        hpw2_scr[...] = jnp.dot(e_ref[...], hp, preferred_element_type=f32) * hpm_ref[...]

    xb = x_ref[...]
    h = jnp.dot(xb, wg_ref[...], preferred_element_type=f32)

    z = jnp.dot(xb, wgm2_scr[...], preferred_element_type=f32) + apvec_scr[...]
    logits = jnp.maximum(z, 0.2 * z)
    m = jnp.max(logits, axis=1, keepdims=True)
    e = jnp.exp(logits - m)
    den = jnp.dot(e, s_ref[...], preferred_element_type=f32) + 1e-16
    rc = jnp.dot(1.0 / den, st_ref[...], preferred_element_type=f32)
    w = e * rc

    agg = jnp.dot(w, hpw2_scr[...], preferred_element_type=f32)
    selfc = jnp.concatenate([w[:, 8:9] * h[:, :C], w[:, 24:25] * h[:, C:]], axis=1)
    gat = agg + selfc + bg_ref[...]

    f0 = jnp.maximum(gat, 0.0)
    f1 = jnp.maximum(jnp.dot(f0, w1_ref[...], preferred_element_type=f32) + b1_ref[...], 0.0)
    f2 = jnp.maximum(jnp.dot(f1, w2_ref[...], preferred_element_type=f32) + b2_ref[...], 0.0)
    f_ref[...] = f2
    preds_ref[...] = jnp.dot(f2, wf_ref[...], preferred_element_type=f32) + bf_ref[...]


def kernel(x, proxies, W_gat, att_src, att_dst, b_gat, W1, b1, W2, b2, Wf, bf):
    grid = (N // BLOCK,)
    row_spec = pl.BlockSpec((BLOCK, EMBED), lambda i: (i, 0))

    def full(shape):
        return pl.BlockSpec(shape, lambda i: tuple(0 for _ in shape))

    preds, f = pl.pallas_call(
        _fused_kernel,
        grid=grid,
        in_specs=[
            row_spec,
            full((P, EMBED)),
            full((EMBED, GOUT)),
            full((1, GOUT)),
            full((1, GOUT)),
            full((1, GOUT)),
            full((4, W)),
            full((P, W)),
            full((P, W)),
            full((1, W)),
            full((W, 8)),
            full((8, W)),
            full((W, P)),
            full((W, GOUT)),
            full((1, GOUT)),
            full((GOUT, HID)),
            full((1, HID)),
            full((HID, EMBED)),
            full((1, EMBED)),
            full((EMBED, OUTD)),
            full((1, OUTD)),
        ],
        out_specs=[pl.BlockSpec((BLOCK, OUTD), lambda i: (i, 0)),
                   pl.BlockSpec((BLOCK, EMBED), lambda i: (i, 0))],
        out_shape=[
            jax.ShapeDtypeStruct((N, OUTD), jnp.float32),
            jax.ShapeDtypeStruct((N, EMBED), jnp.float32),
        ],
        scratch_shapes=[
            pltpu.VMEM((EMBED, W), jnp.float32),
            pltpu.VMEM((W, GOUT), jnp.float32),
            pltpu.VMEM((1, W), jnp.float32),
        ],
    )(
        x,
        proxies,
        W_gat,
        att_src.reshape(1, GOUT),
        att_dst.reshape(1, GOUT),
        jnp.asarray(_M0_np),
        jnp.asarray(_R4_np),
        jnp.asarray(_G0_np),
        jnp.asarray(_G1_np),
        jnp.asarray(_PAD_np),
        jnp.asarray(_S_np),
        jnp.asarray(_ST_np),
        jnp.asarray(_E_np),
        jnp.asarray(_HPmask_np),
        b_gat.reshape(1, GOUT),
        W1,
        b1.reshape(1, HID),
        W2,
        b2.reshape(1, EMBED),
        Wf,
        bf.reshape(1, OUTD),
    )
    return preds, f


# speedup vs baseline: 1.0161x; 1.0161x over previous
import numpy as np
import jax
import jax.numpy as jnp
from jax.experimental import pallas as pl
from jax.experimental.pallas import tpu as pltpu

P = 8
N = 10000
EMBED = 128
H = 2
C = H * EMBED
GOUT = H * C
HID = 4 * EMBED
OUTD = 128

BLOCK = 5000
W = 32
NEG = -1e30

_S_np = np.zeros((W, 8), np.float32)
_S_np[0:9, 0] = 1.0
_S_np[16:25, 1] = 1.0
_ST_np = np.zeros((8, W), np.float32)
_ST_np[0, 0:9] = 1.0
_ST_np[1, 16:25] = 1.0
_E_np = np.zeros((W, P), np.float32)
_E_np[0:8, :] = np.eye(P, dtype=np.float32)
_E_np[16:24, :] = np.eye(P, dtype=np.float32)
_HPmask_np = np.zeros((W, GOUT), np.float32)
_HPmask_np[0:8, :C] = 1.0
_HPmask_np[16:24, C:] = 1.0
_PAD_np = np.full((1, W), NEG, np.float32)
_PAD_np[0, 0:9] = 0.0
_PAD_np[0, 16:25] = 0.0
_G0_np = np.zeros((P, W), np.float32)
_G0_np[:, 0:8] = np.eye(P, dtype=np.float32)
_G1_np = np.zeros((P, W), np.float32)
_G1_np[:, 16:24] = np.eye(P, dtype=np.float32)
_M0_np = np.zeros((1, GOUT), np.float32)
_M0_np[0, :C] = 1.0
_R4_np = np.zeros((4, W), np.float32)
_R4_np[0, 0:8] = 1.0
_R4_np[1, 8] = 1.0
_R4_np[2, 16:24] = 1.0
_R4_np[3, 24] = 1.0


def _fused_kernel(x_ref, p_ref, wg_ref, asrc_ref, adst_ref, m0_ref, r4_ref,
                  g0_ref, g1_ref, pad_ref, s_ref, st_ref, e_ref,
                  hpm_ref, bg_ref,
                  w1_ref, b1_ref, w2_ref, b2_ref, wf_ref, bf_ref,
                  preds_ref, f_ref,
                  wgm2_scr, hpw2_scr, apvec_scr):
    f32 = jnp.float32
    dn = (((1,), (1,)), ((), ()))

    @pl.when(pl.program_id(0) == 0)
    def _prep():
        wg = wg_ref[...]
        asrc = asrc_ref[...]
        adst = adst_ref[...]
        m0 = m0_ref[...]
        m1 = 1.0 - m0
        cols = jnp.concatenate([adst * m0, (asrc + adst) * m0,
                                adst * m1, (asrc + adst) * m1], axis=0)
        w4 = jax.lax.dot_general(wg, cols, dn, preferred_element_type=f32)
        wgm2_scr[...] = jnp.dot(w4, r4_ref[...], preferred_element_type=f32)

        hp = jnp.dot(p_ref[...], wg, preferred_element_type=f32)
        asrc2 = jnp.concatenate([asrc * m0, asrc * m1], axis=0)
        ap2 = jax.lax.dot_general(asrc2, hp, dn, preferred_element_type=f32)
        apvec_scr[...] = (jnp.dot(ap2[0:1], g0_ref[...], preferred_element_type=f32)
                          + jnp.dot(ap2[1:2], g1_ref[...], preferred_element_type=f32)
                          + pad_ref[...])
        hpw2_scr[...] = jnp.dot(e_ref[...], hp, preferred_element_type=f32) * hpm_ref[...]

    xb = x_ref[...]
    h = jnp.dot(xb, wg_ref[...], preferred_element_type=f32)

    z = jnp.dot(xb, wgm2_scr[...], preferred_element_type=f32) + apvec_scr[...]
    logits = jnp.maximum(z, 0.2 * z)
    m = jnp.max(logits, axis=1, keepdims=True)
    e = jnp.exp(logits - m)
    den = jnp.dot(e, s_ref[...], preferred_element_type=f32) + 1e-16
    rc = jnp.dot(1.0 / den, st_ref[...], preferred_element_type=f32)
    w = e * rc

    agg = jnp.dot(w, hpw2_scr[...], preferred_element_type=f32)
    selfc = jnp.concatenate([w[:, 8:9] * h[:, :C], w[:, 24:25] * h[:, C:]], axis=1)
    gat = agg + selfc + bg_ref[...]

    f0 = jnp.maximum(gat, 0.0)
    f1 = jnp.maximum(jnp.dot(f0, w1_ref[...], preferred_element_type=f32) + b1_ref[...], 0.0)
    f2 = jnp.maximum(jnp.dot(f1, w2_ref[...], preferred_element_type=f32) + b2_ref[...], 0.0)
    f_ref[...] = f2
    preds_ref[...] = jnp.dot(f2, wf_ref[...], preferred_element_type=f32) + bf_ref[...]


def kernel(x, proxies, W_gat, att_src, att_dst, b_gat, W1, b1, W2, b2, Wf, bf):
    grid = (N // BLOCK,)
    row_spec = pl.BlockSpec((BLOCK, EMBED), lambda i: (i, 0))

    def full(shape):
        return pl.BlockSpec(shape, lambda i: tuple(0 for _ in shape))

    preds, f = pl.pallas_call(
        _fused_kernel,
        grid=grid,
        in_specs=[
            row_spec,
            full((P, EMBED)),
            full((EMBED, GOUT)),
            full((1, GOUT)),
            full((1, GOUT)),
            full((1, GOUT)),
            full((4, W)),
            full((P, W)),
            full((P, W)),
            full((1, W)),
            full((W, 8)),
            full((8, W)),
            full((W, P)),
            full((W, GOUT)),
            full((1, GOUT)),
            full((GOUT, HID)),
            full((1, HID)),
            full((HID, EMBED)),
            full((1, EMBED)),
            full((EMBED, OUTD)),
            full((1, OUTD)),
        ],
        out_specs=[pl.BlockSpec((BLOCK, OUTD), lambda i: (i, 0)),
                   pl.BlockSpec((BLOCK, EMBED), lambda i: (i, 0))],
        out_shape=[
            jax.ShapeDtypeStruct((N, OUTD), jnp.float32),
            jax.ShapeDtypeStruct((N, EMBED), jnp.float32),
        ],
        scratch_shapes=[
            pltpu.VMEM((EMBED, W), jnp.float32),
            pltpu.VMEM((W, GOUT), jnp.float32),
            pltpu.VMEM((1, W), jnp.float32),
        ],
    )(
        x,
        proxies,
        W_gat,
        att_src.reshape(1, GOUT),
        att_dst.reshape(1, GOUT),
        jnp.asarray(_M0_np),
        jnp.asarray(_R4_np),
        jnp.asarray(_G0_np),
        jnp.asarray(_G1_np),
        jnp.asarray(_PAD_np),
        jnp.asarray(_S_np),
        jnp.asarray(_ST_np),
        jnp.asarray(_E_np),
        jnp.asarray(_HPmask_np),
        b_gat.reshape(1, GOUT),
        W1,
        b1.reshape(1, HID),
        W2,
        b2.reshape(1, EMBED),
        Wf,
        bf.reshape(1, OUTD),
    )
    return preds, f


# speedup vs baseline: 1.0256x; 1.0094x over previous
import numpy as np
import jax
import jax.numpy as jnp
from jax.experimental import pallas as pl
from jax.experimental.pallas import tpu as pltpu

P = 8
N = 10000
EMBED = 128
H = 2
C = H * EMBED
GOUT = H * C
HID = 4 * EMBED
OUTD = 128

BLOCK = 5000
W = 32
NEG = -1e30

_CP_np = np.zeros((40, W), np.float32)
_CP_np[0, 0:8] = 1.0
_CP_np[1, 8] = 1.0
_CP_np[2, 16:24] = 1.0
_CP_np[3, 24] = 1.0
_CP_np[8:16, 0:8] = np.eye(P, dtype=np.float32)
_CP_np[16:24, 16:24] = np.eye(P, dtype=np.float32)
_CP_np[24, :] = NEG
_CP_np[24, 0:9] = 0.0
_CP_np[24, 16:25] = 0.0
_CP_np[32, 0:9] = 1.0
_CP_np[33, 16:25] = 1.0
_SE_np = np.zeros((W, 16), np.float32)
_SE_np[0:9, 0] = 1.0
_SE_np[16:25, 1] = 1.0
_SE_np[0:8, 8:16] = np.eye(P, dtype=np.float32)
_SE_np[16:24, 8:16] = np.eye(P, dtype=np.float32)
_MH_np = np.zeros((40, GOUT), np.float32)
_MH_np[0, :C] = 1.0
_MH_np[8:16, :C] = 1.0
_MH_np[24:32, C:] = 1.0


def _fused_kernel(x_ref, p_ref, wg_ref, asrc_ref, adst_ref, cp_ref, se_ref,
                  mh_ref, bg_ref,
                  w1_ref, b1_ref, w2_ref, b2_ref, wf_ref, bf_ref,
                  preds_ref, f_ref,
                  wgm2_scr, hpw2_scr, apvec_scr):
    f32 = jnp.float32
    dn = (((1,), (1,)), ((), ()))

    @pl.when(pl.program_id(0) == 0)
    def _prep():
        wg = wg_ref[...]
        asrc = asrc_ref[...]
        adst = adst_ref[...]
        m0 = mh_ref[0:1, :]
        m1 = 1.0 - m0
        cols = jnp.concatenate([adst * m0, (asrc + adst) * m0,
                                adst * m1, (asrc + adst) * m1], axis=0)
        w4 = jax.lax.dot_general(wg, cols, dn, preferred_element_type=f32)
        wgm2_scr[...] = jnp.dot(w4, cp_ref[0:4, :], preferred_element_type=f32)

        hp = jnp.dot(p_ref[...], wg, preferred_element_type=f32)
        asrc2 = jnp.concatenate([asrc * m0, asrc * m1], axis=0)
        ap2 = jax.lax.dot_general(asrc2, hp, dn, preferred_element_type=f32)
        apvec_scr[...] = (jnp.dot(ap2[0:1], cp_ref[8:16, :], preferred_element_type=f32)
                          + jnp.dot(ap2[1:2], cp_ref[16:24, :], preferred_element_type=f32)
                          + cp_ref[24:25, :])
        hpw2_scr[...] = (jax.lax.dot_general(se_ref[:, 8:16], hp, (((1,), (0,)), ((), ())),
                                             preferred_element_type=f32) * mh_ref[8:40, :])

    xb = x_ref[...]
    h = jnp.dot(xb, wg_ref[...], preferred_element_type=f32)

    z = jnp.dot(xb, wgm2_scr[...], preferred_element_type=f32) + apvec_scr[...]
    logits = jnp.maximum(z, 0.2 * z)
    m = jnp.max(logits, axis=1, keepdims=True)
    e = jnp.exp(logits - m)
    den = jnp.dot(e, se_ref[:, 0:8], preferred_element_type=f32) + 1e-16
    rc = jnp.dot(1.0 / den, cp_ref[32:40, :], preferred_element_type=f32)
    w = e * rc

    agg = jnp.dot(w, hpw2_scr[...], preferred_element_type=f32)
    selfc = jnp.concatenate([w[:, 8:9] * h[:, :C], w[:, 24:25] * h[:, C:]], axis=1)
    gat = agg + selfc + bg_ref[...]

    f0 = jnp.maximum(gat, 0.0)
    f1 = jnp.maximum(jnp.dot(f0, w1_ref[...], preferred_element_type=f32) + b1_ref[...], 0.0)
    f2 = jnp.maximum(jnp.dot(f1, w2_ref[...], preferred_element_type=f32) + b2_ref[...], 0.0)
    f_ref[...] = f2
    preds_ref[...] = jnp.dot(f2, wf_ref[...], preferred_element_type=f32) + bf_ref[...]


def kernel(x, proxies, W_gat, att_src, att_dst, b_gat, W1, b1, W2, b2, Wf, bf):
    grid = (N // BLOCK,)
    row_spec = pl.BlockSpec((BLOCK, EMBED), lambda i: (i, 0))

    def full(shape):
        return pl.BlockSpec(shape, lambda i: tuple(0 for _ in shape))

    preds, f = pl.pallas_call(
        _fused_kernel,
        grid=grid,
        in_specs=[
            row_spec,
            full((P, EMBED)),
            full((EMBED, GOUT)),
            full((1, GOUT)),
            full((1, GOUT)),
            full((40, W)),
            full((W, 16)),
            full((40, GOUT)),
            full((1, GOUT)),
            full((GOUT, HID)),
            full((1, HID)),
            full((HID, EMBED)),
            full((1, EMBED)),
            full((EMBED, OUTD)),
            full((1, OUTD)),
        ],
        out_specs=[pl.BlockSpec((BLOCK, OUTD), lambda i: (i, 0)),
                   pl.BlockSpec((BLOCK, EMBED), lambda i: (i, 0))],
        out_shape=[
            jax.ShapeDtypeStruct((N, OUTD), jnp.float32),
            jax.ShapeDtypeStruct((N, EMBED), jnp.float32),
        ],
        scratch_shapes=[
            pltpu.VMEM((EMBED, W), jnp.float32),
            pltpu.VMEM((W, GOUT), jnp.float32),
            pltpu.VMEM((1, W), jnp.float32),
        ],
    )(
        x,
        proxies,
        W_gat,
        att_src.reshape(1, GOUT),
        att_dst.reshape(1, GOUT),
        jnp.asarray(_CP_np),
        jnp.asarray(_SE_np),
        jnp.asarray(_MH_np),
        b_gat.reshape(1, GOUT),
        W1,
        b1.reshape(1, HID),
        W2,
        b2.reshape(1, EMBED),
        Wf,
        bf.reshape(1, OUTD),
    )
    return preds, f
